# baseline (speedup 1.0000x reference)
import functools

import jax
import jax.numpy as jnp
from jax import lax
from jax.experimental import pallas as pl
from jax.experimental.pallas import tpu as pltpu
from jax.experimental.pallas import tpu_sc as plsc

NBUF = 4
PITCH = 33


def kernel(input, token_table, pos_table):
    B, S = input.shape
    V, D = token_table.shape
    G = D // 8
    TB = B // 128
    SG = S // 8
    FD = 2 * D

    inp4 = input.reshape(TB, 128, SG, 8).transpose(2, 0, 3, 1)
    tok2 = token_table.reshape(V // 2, FD)

    info = plsc.get_sparse_core_info()
    NC, NS = info.num_cores, info.num_subcores

    mesh = plsc.VectorSubcoreMesh(core_axis_name="c", subcore_axis_name="s")

    @functools.partial(
        pl.kernel,
        mesh=mesh,
        compiler_params=pltpu.CompilerParams(
            use_tc_tiling_on_sc=False, needs_layout_passes=False
        ),
        out_type=jax.ShapeDtypeStruct((S, G, TB, 1024), jnp.float32),
        scratch_types=(
            [pltpu.VMEM((SG, 8, 128), jnp.int32),
             pltpu.VMEM((S, D), jnp.float32)]
            + [pltpu.VMEM((128, FD), jnp.float32) for _ in range(NBUF)]
            + [pltpu.VMEM((128, PITCH), jnp.float32) for _ in range(NBUF)]
            + [pltpu.VMEM((G, 1024), jnp.float32) for _ in range(NBUF)]
            + [pltpu.VMEM((128,), jnp.int32) for _ in range(NBUF)]
            + [pltpu.VMEM((128,), jnp.int32) for _ in range(NBUF)]
            + [pltpu.SemaphoreType.DMA for _ in range(2 * NBUF)]
        ),
    )
    def emb(idx_hbm, tok_hbm, pos_hbm, out_hbm, idx_v, pos_v, *rest):
        bufs = rest[:NBUF]
        pbufs = rest[NBUF:2 * NBUF]
        obufs = rest[2 * NBUF:3 * NBUF]
        fidxs = rest[3 * NBUF:4 * NBUF]
        covs = rest[4 * NBUF:5 * NBUF]
        gsems = rest[5 * NBUF:6 * NBUF]
        osems = rest[6 * NBUF:]

        w = lax.axis_index("s") * NC + lax.axis_index("c")

        pltpu.sync_copy(idx_hbm.at[:, w], idx_v)
        pltpu.sync_copy(pos_hbm, pos_v)

        lanes = jax.lax.broadcasted_iota(jnp.int32, (16,), 0)

        def issue_gather(s, b):
            for k in range(8):
                v = idx_v[s // 8, s % 8, pl.ds(16 * k, 16)]
                fidxs[b][pl.ds(16 * k, 16)] = v // 2
                covs[b][pl.ds(16 * k, 16)] = v % 2
            pltpu.async_copy(tok_hbm.at[fidxs[b]], bufs[b], gsems[b])

        def wait_gather(b):
            pltpu.make_async_copy(
                tok_hbm.at[pl.ds(0, 128)], bufs[b], gsems[b]
            ).wait()

        def compute(s, b):
            buf, pbuf, obuf = bufs[b], pbufs[b], obufs[b]
            cov = covs[b]
            rvecs = [bb0 * 16 + lanes for bb0 in range(8)]
            prow = [pos_v[s, pl.ds(16 * h, 16)] for h in range(2)]

            @plsc.parallel_loop(0, 128, unroll=8)
            def rloop(r):
                chunk = cov[pl.ds((r // 16) * 16, 16)]
                m = chunk.at[jnp.zeros((16,), jnp.int32) + r % 16].get(
                    mode="promise_in_bounds"
                )
                for h in range(2):
                    a0 = buf[r, pl.ds(16 * h, 16)]
                    a1 = buf[r, pl.ds(D + 16 * h, 16)]
                    x = jnp.where(m > 0, a1, a0)
                    pbuf[r, pl.ds(16 * h, 16)] = x + prow[h]

            for h in range(2):
                @plsc.parallel_loop(0, 16, unroll=4)
                def dloop(dq):
                    d = 16 * h + dq
                    cols = jnp.zeros((16,), jnp.int32) + d
                    g = d // 8
                    off = (d % 8) * 128
                    for bb0 in range(8):
                        val = plsc.load_gather(pbuf, [rvecs[bb0], cols])
                        obuf[g, pl.ds(off + bb0 * 16, 16)] = val

        def issue_out(s, b):
            pltpu.async_copy(obufs[b], out_hbm.at[s, :, w], osems[b])

        def wait_out(b):
            pltpu.make_async_copy(
                obufs[b], out_hbm.at[0, :, 0], osems[b]
            ).wait()

        for b in range(NBUF):
            issue_gather(b, b)

        for b in range(NBUF):
            wait_gather(b)
            compute(b, b)
            issue_gather(NBUF + b, b)
            issue_out(b, b)

        T = S // NBUF

        @pl.loop(1, T - 1)
        def outer(t):
            for b in range(NBUF):
                s = t * NBUF + b
                wait_gather(b)
                wait_out(b)
                compute(s, b)
                issue_gather(s + NBUF, b)
                issue_out(s, b)

        for b in range(NBUF):
            s = (T - 1) * NBUF + b
            wait_gather(b)
            wait_out(b)
            compute(s, b)
            issue_out(s, b)

        for b in range(NBUF):
            wait_out(b)

    out5 = emb(inp4, tok2, pos_table)
    out = out5.reshape(S, G, TB, 8, 128).transpose(2, 4, 0, 1, 3)
    return out.reshape(B, S, D)


# speedup vs baseline: 1.1787x; 1.1787x over previous
import functools

import jax
import jax.numpy as jnp
from jax import lax
from jax.experimental import pallas as pl
from jax.experimental.pallas import tpu as pltpu
from jax.experimental.pallas import tpu_sc as plsc

NBUF = 4
PITCH = 33


def _relayout(token_table):
    V, D = token_table.shape
    NPAIR = V // 256
    VMAIN = NPAIR * 256
    tokT = token_table.T
    aux = token_table[VMAIN:].reshape(16, 128)

    mesh = plsc.VectorSubcoreMesh(core_axis_name="c", subcore_axis_name="s")

    @functools.partial(
        pl.kernel,
        mesh=mesh,
        compiler_params=pltpu.CompilerParams(
            use_tc_tiling_on_sc=True, needs_layout_passes=False
        ),
        out_type=jax.ShapeDtypeStruct((V // 4, 128), jnp.float32),
        scratch_types=[
            pltpu.VMEM((2, D, 256), jnp.float32),
            pltpu.VMEM((D, 257), jnp.float32),
            pltpu.VMEM((8, 128), jnp.float32),
            pltpu.VMEM((16, 128), jnp.float32),
            pltpu.SemaphoreType.DMA,
            pltpu.SemaphoreType.DMA,
        ],
    )
    def relayout(tokT_hbm, aux_hbm, out_hbm, tbuf, pbuf, obuf, xbuf,
                 sem0, sem1):
        w = lax.axis_index("s") * 2 + lax.axis_index("c")
        lanes = jax.lax.broadcasted_iota(jnp.int32, (16,), 0)
        sems = [sem0, sem1]

        nbase = 122 * w

        def issue(k, sl):
            pltpu.async_copy(
                tokT_hbm.at[:, pl.ds(k * 256, 256)], tbuf.at[sl], sems[sl]
            )

        def wait(sl):
            pltpu.make_async_copy(
                tokT_hbm.at[:, pl.ds(0, 256)], tbuf.at[sl], sems[sl]
            ).wait()

        def process(k, sl):
            tb = tbuf.at[sl]

            @plsc.parallel_loop(0, D, unroll=4)
            def repitch(d):
                for q in range(16):
                    pbuf[d, pl.ds(16 * q, 16)] = tb[d, pl.ds(16 * q, 16)]

            @plsc.parallel_loop(0, 256, unroll=8)
            def tr(v):
                r = v // 4
                coff = (v % 4) * 32
                cols = jnp.zeros((16,), jnp.int32) + v
                for h in range(2):
                    obuf[r, pl.ds(coff + 16 * h, 16)] = plsc.load_gather(
                        pbuf, [lanes + 16 * h, cols]
                    )

            pltpu.sync_copy(obuf, out_hbm.at[pl.ds(k * 8, 8)])

        issue(nbase, 0)
        issue(nbase + 1, 1)

        @pl.loop(0, 60)
        def cloop(j):
            k = nbase + 2 * j
            wait(0)
            process(k, 0)
            issue(k + 2, 0)
            wait(1)
            process(k + 1, 1)
            issue(k + 3, 1)

        wait(0)
        process(nbase + 120, 0)
        wait(1)
        process(nbase + 121, 1)

        @pl.when(w < 2)
        def _extra():
            issue(32 * 122 + w, 0)
            wait(0)
            process(32 * 122 + w, 0)

        @pl.when(w == 0)
        def _tail():
            pltpu.sync_copy(aux_hbm, xbuf)
            pltpu.sync_copy(xbuf, out_hbm.at[pl.ds(VMAIN // 4, 16)])

    return relayout(tokT, aux)


def kernel(input, token_table, pos_table):
    B, S = input.shape
    V, D = token_table.shape
    G = D // 8
    TB = B // 128
    SG = S // 8

    inp4 = input.reshape(TB, 128, SG, 8).transpose(2, 0, 3, 1)

    tok_lin = _relayout(token_table).reshape(V, D)

    info = plsc.get_sparse_core_info()
    NC, NS = info.num_cores, info.num_subcores

    mesh = plsc.VectorSubcoreMesh(core_axis_name="c", subcore_axis_name="s")

    @functools.partial(
        pl.kernel,
        mesh=mesh,
        compiler_params=pltpu.CompilerParams(
            use_tc_tiling_on_sc=False, needs_layout_passes=False
        ),
        out_type=jax.ShapeDtypeStruct((S, G, TB, 1024), jnp.float32),
        scratch_types=(
            [pltpu.VMEM((SG, 8, 128), jnp.int32),
             pltpu.VMEM((S, D), jnp.float32)]
            + [pltpu.VMEM((128, D), jnp.float32) for _ in range(NBUF)]
            + [pltpu.VMEM((128, PITCH), jnp.float32) for _ in range(NBUF)]
            + [pltpu.VMEM((G, 1024), jnp.float32) for _ in range(NBUF)]
            + [pltpu.SemaphoreType.DMA for _ in range(2 * NBUF)]
        ),
    )
    def emb(idx_hbm, tok_hbm, pos_hbm, out_hbm, idx_v, pos_v, *rest):
        bufs = rest[:NBUF]
        pbufs = rest[NBUF:2 * NBUF]
        obufs = rest[2 * NBUF:3 * NBUF]
        gsems = rest[3 * NBUF:4 * NBUF]
        osems = rest[4 * NBUF:]

        w = lax.axis_index("s") * NC + lax.axis_index("c")

        pltpu.sync_copy(idx_hbm.at[:, w], idx_v)
        pltpu.sync_copy(pos_hbm, pos_v)

        lanes = jax.lax.broadcasted_iota(jnp.int32, (16,), 0)

        def issue_gather(s, b):
            pltpu.async_copy(
                tok_hbm.at[idx_v.at[s // 8, s % 8]], bufs[b], gsems[b]
            )

        def wait_gather(b):
            pltpu.make_async_copy(
                tok_hbm.at[pl.ds(0, 128)], bufs[b], gsems[b]
            ).wait()

        def compute(s, b):
            buf, pbuf, obuf = bufs[b], pbufs[b], obufs[b]
            rvecs = [bb0 * 16 + lanes for bb0 in range(8)]
            prow = [pos_v[s, pl.ds(16 * h, 16)] for h in range(2)]

            @plsc.parallel_loop(0, 128, unroll=8)
            def rloop(r):
                for h in range(2):
                    pbuf[r, pl.ds(16 * h, 16)] = (
                        buf[r, pl.ds(16 * h, 16)] + prow[h]
                    )

            for h in range(2):
                @plsc.parallel_loop(0, 16, unroll=4)
                def dloop(dq):
                    d = 16 * h + dq
                    cols = jnp.zeros((16,), jnp.int32) + d
                    g = d // 8
                    off = (d % 8) * 128
                    for bb0 in range(8):
                        val = plsc.load_gather(pbuf, [rvecs[bb0], cols])
                        obuf[g, pl.ds(off + bb0 * 16, 16)] = val

        def issue_out(s, b):
            pltpu.async_copy(obufs[b], out_hbm.at[s, :, w], osems[b])

        def wait_out(b):
            pltpu.make_async_copy(
                obufs[b], out_hbm.at[0, :, 0], osems[b]
            ).wait()

        for b in range(NBUF):
            issue_gather(b, b)

        for b in range(NBUF):
            wait_gather(b)
            compute(b, b)
            issue_gather(NBUF + b, b)
            issue_out(b, b)

        T = S // NBUF

        @pl.loop(1, T - 1)
        def outer(t):
            for b in range(NBUF):
                s = t * NBUF + b
                wait_gather(b)
                wait_out(b)
                compute(s, b)
                issue_gather(s + NBUF, b)
                issue_out(s, b)

        for b in range(NBUF):
            s = (T - 1) * NBUF + b
            wait_gather(b)
            wait_out(b)
            compute(s, b)
            issue_out(s, b)

        for b in range(NBUF):
            wait_out(b)

    out5 = emb(inp4, tok_lin, pos_table)
    out = out5.reshape(S, G, TB, 8, 128).transpose(2, 4, 0, 1, 3)
    return out.reshape(B, S, D)
